# baseline (speedup 1.0000x reference)
import functools

import jax
import jax.numpy as jnp
from jax import lax
from jax.experimental import pallas as pl
from jax.experimental.pallas import tpu as pltpu
from jax.experimental.pallas import tpu_sc as plsc

N = 10000
D = 128
H = 128
G = 256
K = 6
E = 320000

NC = 2
NS = 16
NW = NC * NS
CH = 128
NBUF = 1
IB = 8
NBLK = -(-E // (NW * CH * IB * 2)) * 2
CHUNKS = NBLK * IB
E_PAD = NW * CH * CHUNKS
N_DUMMY = N
N_PAD = ((N + 1 + 127) // 128) * 128
ZROWS = N_PAD // NS


def _sc_aggregate(h, src_p, dst_p, zeros_acc):
    mesh = plsc.VectorSubcoreMesh(core_axis_name="c", subcore_axis_name="s")

    @functools.partial(
        pl.kernel,
        out_type=jax.ShapeDtypeStruct((NC, N_PAD, H), jnp.float32),
        mesh=mesh,
        scratch_types=[
            pltpu.VMEM((2, IB, CH), jnp.int32),
            pltpu.VMEM((2, IB, CH), jnp.int32),
            pltpu.VMEM((NBUF, CH, H), jnp.float32),
            pltpu.VMEM_SHARED((N_PAD, H), jnp.float32),
        ] + [pltpu.SemaphoreType.DMA] * (NBUF + 2),
    )
    def agg(h_hbm, src_hbm, dst_hbm, z_hbm, out_hbm, sidx_v, didx_v, rows_v,
            acc_sh, *sems):
        gsem = sems[:NBUF]
        isem = sems[NBUF:]
        cid = lax.axis_index("c")
        sid = lax.axis_index("s")
        wid = cid * NS + sid

        pltpu.sync_copy(z_hbm.at[pl.ds(sid * ZROWS, ZROWS)],
                        acc_sh.at[pl.ds(sid * ZROWS, ZROWS)])
        for p in range(2):
            pltpu.async_copy(src_hbm.at[wid, pl.ds(p * IB, IB)],
                             sidx_v.at[p], isem[p])
            pltpu.async_copy(dst_hbm.at[wid, pl.ds(p * IB, IB)],
                             didx_v.at[p], isem[p])
        plsc.subcore_barrier()

        @pl.loop(0, NBLK, step=2)
        def _blk(k0):
            for p in range(2):
                k = k0 + p
                pltpu.make_async_copy(src_hbm.at[wid, pl.ds(k * IB, IB)],
                                      sidx_v.at[p], isem[p]).wait()
                pltpu.make_async_copy(dst_hbm.at[wid, pl.ds(k * IB, IB)],
                                      didx_v.at[p], isem[p]).wait()
                for b in range(NBUF):
                    pltpu.async_copy(h_hbm.at[sidx_v.at[p, b]],
                                     rows_v.at[b], gsem[b])
                for t in range(IB):
                    b = t % NBUF
                    pltpu.make_async_copy(h_hbm.at[sidx_v.at[p, t]],
                                          rows_v.at[b], gsem[b]).wait()
                    pltpu.sync_copy(rows_v.at[b], acc_sh.at[didx_v.at[p, t]],
                                    add=True)
                    if t + NBUF < IB:
                        pltpu.async_copy(h_hbm.at[sidx_v.at[p, t + NBUF]],
                                         rows_v.at[b], gsem[b])
                @pl.when(k + 2 < NBLK)
                def _():
                    pltpu.async_copy(src_hbm.at[wid, pl.ds((k + 2) * IB, IB)],
                                     sidx_v.at[p], isem[p])
                    pltpu.async_copy(dst_hbm.at[wid, pl.ds((k + 2) * IB, IB)],
                                     didx_v.at[p], isem[p])

        plsc.subcore_barrier()

        pltpu.sync_copy(acc_sh.at[pl.ds(sid * ZROWS, ZROWS)],
                        out_hbm.at[cid, pl.ds(sid * ZROWS, ZROWS)])

    return agg(h, src_p, dst_p, zeros_acc)


def _r(x):
    return x.astype(jnp.bfloat16).astype(jnp.float32)


def _hdot(a, b):
    return jnp.dot(a, b, preferred_element_type=jnp.float32,
                   precision=lax.Precision.HIGHEST)


def _tc_first(x_pad, w_nbr0):
    def body(x_ref, wn_ref, o_ref):
        o_ref[...] = _hdot(_r(x_ref[...]), _r(wn_ref[...]))

    return pl.pallas_call(
        body,
        out_shape=jax.ShapeDtypeStruct((N_PAD, H), jnp.float32),
    )(x_pad, w_nbr0)


def _tc_layer(acc2, h, w_self, b, w_nbr_next):
    def body(acc_ref, h_ref, ws_ref, b_ref, wn_ref, h_out, hw_out):
        agg = acc_ref[0] + acc_ref[1]
        o = jnp.maximum(
            agg + _hdot(_r(h_ref[...]), _r(ws_ref[...])) + b_ref[...], 0.0)
        row = lax.broadcasted_iota(jnp.int32, (N_PAD, H), 0)
        o = jnp.where(row < N, o, 0.0)
        h_out[...] = o
        hw_out[...] = _hdot(_r(o), _r(wn_ref[...]))

    return pl.pallas_call(
        body,
        out_shape=(jax.ShapeDtypeStruct((N_PAD, H), jnp.float32),
                   jax.ShapeDtypeStruct((N_PAD, H), jnp.float32)),
    )(acc2, h, w_self, b, w_nbr_next)


def _tc_final(acc2, h, w_self, b, att_w, att_b, batch2d, heads_w8, heads_b8):
    def body(acc_ref, h_ref, ws_ref, b_ref, aw_ref, ab_ref, batch_ref,
             hw_ref, hb_ref, o_ref):
        agg = acc_ref[0] + acc_ref[1]
        hv = jnp.maximum(
            agg + _hdot(_r(h_ref[...]), _r(ws_ref[...])) + b_ref[...], 0.0)
        row = lax.broadcasted_iota(jnp.int32, (N_PAD, H), 0)
        hv = jnp.where(row < N, hv, 0.0)
        s = _hdot(_r(hv), _r(aw_ref[...]))[:, 0] + ab_ref[0, 0]
        batch = batch_ref[0]
        valid = batch < G
        gids = lax.broadcasted_iota(jnp.int32, (G, N_PAD), 0)
        onehot = batch[None, :] == gids
        neg = jnp.float32(-jnp.inf)
        smax = jnp.max(jnp.where(onehot, s[None, :], neg), axis=1)
        smax_n = jnp.sum(jnp.where(onehot, smax[:, None], 0.0), axis=0)
        ex = jnp.where(valid, jnp.exp(s - smax_n), 0.0)
        denom = jnp.sum(jnp.where(onehot, ex[None, :], 0.0), axis=1)
        denom_n = jnp.sum(jnp.where(onehot, denom[:, None], 0.0), axis=0)
        alpha = jnp.where(valid, ex / jnp.maximum(denom_n, 1e-12), 0.0)
        m = onehot.astype(jnp.float32)
        pooled = _hdot(m, alpha[:, None] * hv)
        o_ref[...] = _hdot(_r(hw_ref[...]), _r(pooled).T) + hb_ref[...]

    return pl.pallas_call(
        body,
        out_shape=jax.ShapeDtypeStruct((8, G), jnp.float32),
    )(acc2, h, w_self, b, att_w, att_b, batch2d, heads_w8, heads_b8)


def kernel(x, edge_index, batch, params):
    src, dst = edge_index[0], edge_index[1]
    pad = E_PAD - E
    src_p = jnp.concatenate(
        [src, jnp.zeros((pad,), jnp.int32)]).reshape(NW, CHUNKS, CH)
    pad_dst = N_DUMMY + jnp.arange(pad, dtype=jnp.int32) % (N_PAD - N_DUMMY)
    dst_p = jnp.concatenate([dst, pad_dst]).reshape(NW, CHUNKS, CH)
    zeros_acc = jnp.zeros((N_PAD, H), jnp.float32)

    layers = params["layers"]
    h = jnp.zeros((N_PAD, H), jnp.float32).at[:N].set(x)
    hw = _tc_first(h, layers[0]["W_nbr"])
    for i in range(len(layers) - 1):
        acc2 = _sc_aggregate(hw, src_p, dst_p, zeros_acc)
        h, hw = _tc_layer(acc2, h, layers[i]["W_self"],
                          layers[i]["b"].reshape(1, H),
                          layers[i + 1]["W_nbr"])
    acc2 = _sc_aggregate(hw, src_p, dst_p, zeros_acc)

    att_w = params["att_w"].reshape(H, 1)
    att_b = params["att_b"].reshape(1, 1)
    batch2d = jnp.full((1, N_PAD), G, jnp.int32).at[0, :N].set(batch)
    heads_w8 = jnp.zeros((8, H), jnp.float32).at[:K].set(params["heads_W"])
    heads_b8 = jnp.zeros((8, 1), jnp.float32).at[:K, 0].set(params["heads_b"])
    out8 = _tc_final(acc2, h, layers[-1]["W_self"],
                     layers[-1]["b"].reshape(1, H),
                     att_w, att_b, batch2d, heads_w8, heads_b8)
    return out8[:K]


# speedup vs baseline: 1.4273x; 1.4273x over previous
import functools

import jax
import jax.numpy as jnp
from jax import lax
from jax.experimental import pallas as pl
from jax.experimental.pallas import tpu as pltpu
from jax.experimental.pallas import tpu_sc as plsc

N = 10000
D = 128
H = 128
G = 256
K = 6
E = 320000

NC = 2
NS = 16
NW = NC * NS
CH = 96
NBUF = 2
CHUNKS = -(-E // (NW * CH * NBUF)) * NBUF
E_PAD = NW * CH * CHUNKS
E_TILE = CHUNKS * CH
N_DUMMY = N
N_PAD = ((N + 1 + 127) // 128) * 128
ZROWS = N_PAD // NS


def _sc_aggregate(h, src_p, dst_p, zeros_acc):
    mesh = plsc.VectorSubcoreMesh(core_axis_name="c", subcore_axis_name="s")

    @functools.partial(
        pl.kernel,
        out_type=jax.ShapeDtypeStruct((NC, N_PAD, H), jnp.float32),
        mesh=mesh,
        scratch_types=[
            pltpu.VMEM((E_TILE,), jnp.int32),
            pltpu.VMEM((CHUNKS, CH), jnp.int32),
            pltpu.VMEM((NBUF, CH, H), jnp.float32),
            pltpu.VMEM_SHARED((N_PAD, H), jnp.float32),
        ] + [pltpu.SemaphoreType.DMA] * NBUF,
    )
    def agg(h_hbm, src_hbm, dst_hbm, z_hbm, out_hbm, sidx_v, didx_v, rows_v,
            acc_sh, *gsem):
        cid = lax.axis_index("c")
        sid = lax.axis_index("s")
        wid = cid * NS + sid

        pltpu.sync_copy(z_hbm.at[pl.ds(sid * ZROWS, ZROWS)],
                        acc_sh.at[pl.ds(sid * ZROWS, ZROWS)])
        pltpu.sync_copy(src_hbm.at[wid], sidx_v)
        pltpu.sync_copy(dst_hbm.at[wid], didx_v)
        plsc.subcore_barrier()

        for b in range(NBUF):
            pltpu.async_copy(h_hbm.at[sidx_v.at[pl.ds(b * CH, CH)]],
                             rows_v.at[b], gsem[b])

        @pl.loop(0, CHUNKS, step=NBUF)
        def _chunk(j0):
            for b in range(NBUF):
                j = j0 + b
                pltpu.make_async_copy(
                    h_hbm.at[sidx_v.at[pl.ds(j * CH, CH)]],
                    rows_v.at[b], gsem[b]).wait()
                pltpu.sync_copy(rows_v.at[b], acc_sh.at[didx_v.at[j]],
                                add=True)

                @pl.when(j + NBUF < CHUNKS)
                def _():
                    pltpu.async_copy(
                        h_hbm.at[sidx_v.at[pl.ds((j + NBUF) * CH, CH)]],
                        rows_v.at[b], gsem[b])

        plsc.subcore_barrier()

        pltpu.sync_copy(acc_sh.at[pl.ds(sid * ZROWS, ZROWS)],
                        out_hbm.at[cid, pl.ds(sid * ZROWS, ZROWS)])

    return agg(h, src_p, dst_p, zeros_acc)


def _r(x):
    return x.astype(jnp.bfloat16).astype(jnp.float32)


def _hdot(a, b):
    return jnp.dot(a, b, preferred_element_type=jnp.float32,
                   precision=lax.Precision.HIGHEST)


def _tc_first(x_pad, w_nbr0):
    def body(x_ref, wn_ref, o_ref):
        o_ref[...] = _hdot(_r(x_ref[...]), _r(wn_ref[...]))

    return pl.pallas_call(
        body,
        out_shape=jax.ShapeDtypeStruct((N_PAD, H), jnp.float32),
    )(x_pad, w_nbr0)


def _tc_layer(acc2, h, w_self, b, w_nbr_next):
    def body(acc_ref, h_ref, ws_ref, b_ref, wn_ref, h_out, hw_out):
        agg = acc_ref[0] + acc_ref[1]
        o = jnp.maximum(
            agg + _hdot(_r(h_ref[...]), _r(ws_ref[...])) + b_ref[...], 0.0)
        row = lax.broadcasted_iota(jnp.int32, (N_PAD, H), 0)
        o = jnp.where(row < N, o, 0.0)
        h_out[...] = o
        hw_out[...] = _hdot(_r(o), _r(wn_ref[...]))

    return pl.pallas_call(
        body,
        out_shape=(jax.ShapeDtypeStruct((N_PAD, H), jnp.float32),
                   jax.ShapeDtypeStruct((N_PAD, H), jnp.float32)),
    )(acc2, h, w_self, b, w_nbr_next)


def _tc_final(acc2, h, w_self, b, att_w, att_b, batch2d, heads_w8, heads_b8):
    def body(acc_ref, h_ref, ws_ref, b_ref, aw_ref, ab_ref, batch_ref,
             hw_ref, hb_ref, o_ref):
        agg = acc_ref[0] + acc_ref[1]
        hv = jnp.maximum(
            agg + _hdot(_r(h_ref[...]), _r(ws_ref[...])) + b_ref[...], 0.0)
        row = lax.broadcasted_iota(jnp.int32, (N_PAD, H), 0)
        hv = jnp.where(row < N, hv, 0.0)
        s = _hdot(_r(hv), _r(aw_ref[...]))[:, 0] + ab_ref[0, 0]
        batch = batch_ref[0]
        valid = batch < G
        gids = lax.broadcasted_iota(jnp.int32, (G, N_PAD), 0)
        onehot = batch[None, :] == gids
        neg = jnp.float32(-jnp.inf)
        smax = jnp.max(jnp.where(onehot, s[None, :], neg), axis=1)
        smax_n = jnp.sum(jnp.where(onehot, smax[:, None], 0.0), axis=0)
        ex = jnp.where(valid, jnp.exp(s - smax_n), 0.0)
        denom = jnp.sum(jnp.where(onehot, ex[None, :], 0.0), axis=1)
        denom_n = jnp.sum(jnp.where(onehot, denom[:, None], 0.0), axis=0)
        alpha = jnp.where(valid, ex / jnp.maximum(denom_n, 1e-12), 0.0)
        m = onehot.astype(jnp.float32)
        pooled = _hdot(m, alpha[:, None] * hv)
        o_ref[...] = _hdot(_r(hw_ref[...]), _r(pooled).T) + hb_ref[...]

    return pl.pallas_call(
        body,
        out_shape=jax.ShapeDtypeStruct((8, G), jnp.float32),
    )(acc2, h, w_self, b, att_w, att_b, batch2d, heads_w8, heads_b8)


def kernel(x, edge_index, batch, params):
    src, dst = edge_index[0], edge_index[1]
    pad = E_PAD - E
    src_p = jnp.concatenate(
        [src, jnp.zeros((pad,), jnp.int32)]).reshape(NW, E_TILE)
    pad_dst = N_DUMMY + jnp.arange(pad, dtype=jnp.int32) % (N_PAD - N_DUMMY)
    dst_p = jnp.concatenate([dst, pad_dst]).reshape(NW, CHUNKS, CH)
    zeros_acc = jnp.zeros((N_PAD, H), jnp.float32)

    layers = params["layers"]
    h = jnp.zeros((N_PAD, H), jnp.float32).at[:N].set(x)
    hw = _tc_first(h, layers[0]["W_nbr"])
    for i in range(len(layers) - 1):
        acc2 = _sc_aggregate(hw, src_p, dst_p, zeros_acc)
        h, hw = _tc_layer(acc2, h, layers[i]["W_self"],
                          layers[i]["b"].reshape(1, H),
                          layers[i + 1]["W_nbr"])
    acc2 = _sc_aggregate(hw, src_p, dst_p, zeros_acc)

    att_w = params["att_w"].reshape(H, 1)
    att_b = params["att_b"].reshape(1, 1)
    batch2d = jnp.full((1, N_PAD), G, jnp.int32).at[0, :N].set(batch)
    heads_w8 = jnp.zeros((8, H), jnp.float32).at[:K].set(params["heads_W"])
    heads_b8 = jnp.zeros((8, 1), jnp.float32).at[:K, 0].set(params["heads_b"])
    out8 = _tc_final(acc2, h, layers[-1]["W_self"],
                     layers[-1]["b"].reshape(1, H),
                     att_w, att_b, batch2d, heads_w8, heads_b8)
    return out8[:K]
